# baseline (speedup 1.0000x reference)
import functools

import jax
import jax.numpy as jnp
from jax import lax
from jax.experimental import pallas as pl
from jax.experimental.pallas import tpu as pltpu
from jax.experimental.pallas import tpu_sc as plsc

E = 64
AUX_W = 0.01
T, C, DFF = 2048, 768, 3072
DFF_CHUNK = 512
ND = DFF // DFF_CHUNK
BM = 128
NB = T // BM * 2 + E
S = NB * BM
NW = 32
TPW = T // NW
CB = 256


def _router_body(x_ref, gate_ref, posa_ref, posb_ref, wa_ref, wb_ref,
                 sched_ref, aux_ref):
    x = x_ref[...]
    logits = lax.dot_general(x, gate_ref[...], (((1,), (1,)), ((), ())),
                             preferred_element_type=jnp.float32)
    lane = lax.broadcasted_iota(jnp.int32, (T, E), 1)
    m1 = jnp.max(logits, axis=1, keepdims=True)
    i1 = jnp.min(jnp.where(logits == m1, lane, E), axis=1, keepdims=True)
    masked = jnp.where(lane == i1, -jnp.inf, logits)
    m2 = jnp.max(masked, axis=1, keepdims=True)
    i2 = jnp.min(jnp.where(masked == m2, lane, E), axis=1, keepdims=True)
    t = jnp.exp(m2 - m1)
    wa_ref[...] = jnp.broadcast_to(1.0 / (1.0 + t), (T, 16))
    wb_ref[...] = jnp.broadcast_to(t / (1.0 + t), (T, 16))
    p = jnp.exp(logits - m1)
    p = p / jnp.sum(p, axis=1, keepdims=True)
    frac = jnp.mean(p, axis=0, keepdims=True)
    aux_ref[...] = (AUX_W * E * jnp.sum(frac * frac)).reshape(1, 1)

    oh1 = jnp.where(lane == i1, 1.0, 0.0)
    oh2 = jnp.where(lane == i2, 1.0, 0.0)
    ohsum = oh1 + oh2
    rsub = lax.broadcasted_iota(jnp.int32, (CB, CB), 0)
    csub = lax.broadcasted_iota(jnp.int32, (CB, CB), 1)
    tril = jnp.where(csub < rsub, 1.0, 0.0)
    exc_blocks = []
    base = jnp.zeros((1, E), dtype=jnp.float32)
    for rb in range(T // CB):
        mb = ohsum[rb * CB:(rb + 1) * CB, :]
        exc_blocks.append(
            lax.dot_general(tril, mb, (((1,), (0,)), ((), ())),
                            preferred_element_type=jnp.float32) + base)
        base = base + jnp.sum(mb, axis=0, keepdims=True)
    exc = jnp.concatenate(exc_blocks, axis=0)
    counts = base
    nb = jnp.ceil(counts * (1.0 / BM))
    ce1 = lax.broadcasted_iota(jnp.int32, (E, E), 0)
    ce2 = lax.broadcasted_iota(jnp.int32, (E, E), 1)
    upper = jnp.where(ce1 < ce2, 1.0, 0.0)
    bstart = lax.dot_general(nb, upper, (((1,), (0,)), ((), ())),
                             preferred_element_type=jnp.float32)
    sbase = bstart * BM
    posa = jnp.sum(oh1 * (sbase + exc), axis=1, keepdims=True)
    posb = jnp.sum(oh2 * (sbase + exc), axis=1, keepdims=True)
    posa_ref[...] = posa.astype(jnp.int32)
    posb_ref[...] = posb.astype(jnp.int32)
    biota = lax.broadcasted_iota(jnp.int32, (NB, 1), 0).astype(jnp.float32)
    sched = jnp.sum(jnp.where(bstart <= biota, 1.0, 0.0),
                    axis=1, keepdims=True) - 1.0
    total = jnp.sum(nb)
    lane_e = lax.broadcasted_iota(jnp.int32, (1, E), 1).astype(jnp.float32)
    last_used = jnp.max(jnp.where(nb > 0, lane_e, -1.0))
    sched = jnp.where(biota < total, sched, last_used)
    sched_ref[...] = sched.astype(jnp.int32)


def _router(xr, gate_w):
    return pl.pallas_call(
        _router_body,
        grid=(1,),
        in_specs=[
            pl.BlockSpec((T, C), lambda i: (0, 0)),
            pl.BlockSpec((E, C), lambda i: (0, 0)),
        ],
        out_specs=[
            pl.BlockSpec((T, 1), lambda i: (0, 0)),
            pl.BlockSpec((T, 1), lambda i: (0, 0)),
            pl.BlockSpec((T, 16), lambda i: (0, 0)),
            pl.BlockSpec((T, 16), lambda i: (0, 0)),
            pl.BlockSpec((NB, 1), lambda i: (0, 0)),
            pl.BlockSpec((1, 1), lambda i: (0, 0)),
        ],
        out_shape=[
            jax.ShapeDtypeStruct((T, 1), jnp.int32),
            jax.ShapeDtypeStruct((T, 1), jnp.int32),
            jax.ShapeDtypeStruct((T, 16), jnp.float32),
            jax.ShapeDtypeStruct((T, 16), jnp.float32),
            jax.ShapeDtypeStruct((NB, 1), jnp.int32),
            jax.ShapeDtypeStruct((1, 1), jnp.float32),
        ],
    )(xr, gate_w)


@functools.lru_cache(maxsize=1)
def _sc_kernels():
    mesh = plsc.VectorSubcoreMesh(core_axis_name="c", subcore_axis_name="s")

    @functools.partial(
        pl.kernel,
        mesh=mesh,
        out_type=jax.ShapeDtypeStruct((S, C), jnp.float32),
        scratch_types=[
            pltpu.VMEM((TPW,), jnp.int32),
            pltpu.VMEM((TPW,), jnp.int32),
            pltpu.VMEM((TPW, C), jnp.float32),
            pltpu.SemaphoreType.DMA,
            pltpu.SemaphoreType.DMA,
        ],
    )
    def _sc_dispatch(xr_hbm, posa_hbm, posb_hbm, xs_hbm,
                     posa_v, posb_v, rows_v, sema, semb):
        wid = lax.axis_index("s") * 2 + lax.axis_index("c")
        base = wid * TPW
        pltpu.sync_copy(posa_hbm.at[pl.ds(base, TPW)], posa_v)
        pltpu.sync_copy(posb_hbm.at[pl.ds(base, TPW)], posb_v)
        pltpu.sync_copy(xr_hbm.at[pl.ds(base, TPW)], rows_v)
        cpa = pltpu.async_copy(rows_v, xs_hbm.at[posa_v], sema)
        cpb = pltpu.async_copy(rows_v, xs_hbm.at[posb_v], semb)
        cpa.wait()
        cpb.wait()

    @functools.partial(
        pl.kernel,
        mesh=mesh,
        out_type=jax.ShapeDtypeStruct((T, C), jnp.float32),
        scratch_types=[
            pltpu.VMEM((TPW,), jnp.int32),
            pltpu.VMEM((TPW,), jnp.int32),
            pltpu.VMEM((TPW, 16), jnp.float32),
            pltpu.VMEM((TPW, 16), jnp.float32),
            pltpu.VMEM((TPW, C), jnp.float32),
            pltpu.VMEM((TPW, C), jnp.float32),
            pltpu.SemaphoreType.DMA,
            pltpu.SemaphoreType.DMA,
        ],
    )
    def _sc_combine(ys_hbm, posa_hbm, posb_hbm, wa_hbm, wb_hbm, out_hbm,
                    posa_v, posb_v, wa_v, wb_v, rowsa_v, rowsb_v, sema, semb):
        wid = lax.axis_index("s") * 2 + lax.axis_index("c")
        base = wid * TPW
        pltpu.sync_copy(posa_hbm.at[pl.ds(base, TPW)], posa_v)
        pltpu.sync_copy(posb_hbm.at[pl.ds(base, TPW)], posb_v)
        pltpu.sync_copy(wa_hbm.at[pl.ds(base, TPW)], wa_v)
        pltpu.sync_copy(wb_hbm.at[pl.ds(base, TPW)], wb_v)
        cpa = pltpu.async_copy(ys_hbm.at[posa_v], rowsa_v, sema)
        cpb = pltpu.async_copy(ys_hbm.at[posb_v], rowsb_v, semb)
        cpa.wait()
        cpb.wait()

        def tok_body(ti, carry):
            wa16 = wa_v[ti, :]
            wb16 = wb_v[ti, :]
            for c in range(C // 16):
                ra = rowsa_v[ti, pl.ds(c * 16, 16)]
                rb = rowsb_v[ti, pl.ds(c * 16, 16)]
                rowsa_v[ti, pl.ds(c * 16, 16)] = wa16 * ra + wb16 * rb
            return carry

        lax.fori_loop(0, TPW, tok_body, 0)
        pltpu.sync_copy(rowsa_v, out_hbm.at[pl.ds(base, TPW)])

    return _sc_dispatch, _sc_combine


_NS = 2
_CH = C // _NS
_DH = DFF // _NS


def _ffn_body(sched_ref, xs_ref, *refs):
    wf_refs = refs[:_NS]
    wp_refs = refs[_NS:2 * _NS]
    ys_ref = refs[2 * _NS]
    x = xs_ref[...]
    h = sum(
        lax.dot_general(x[:, q * _CH:(q + 1) * _CH], wf_refs[q][0],
                        (((1,), (0,)), ((), ())),
                        preferred_element_type=jnp.float32)
        for q in range(_NS))
    h = 0.5 * h * (1.0 + lax.erf(h * 0.7071067811865476))
    ys_ref[...] = sum(
        lax.dot_general(h[:, q * _DH:(q + 1) * _DH], wp_refs[q][0],
                        (((1,), (0,)), ((), ())),
                        preferred_element_type=jnp.float32)
        for q in range(_NS))


def _make_wf_spec(q):
    return pl.BlockSpec((1, _CH, DFF), lambda b, sched: (sched[b], q, 0))


def _make_wp_spec(q):
    return pl.BlockSpec((1, _DH, C), lambda b, sched: (sched[b], q, 0))


def _ffn(sched, xs, w_fc, w_proj):
    grid_spec = pltpu.PrefetchScalarGridSpec(
        num_scalar_prefetch=1,
        grid=(NB,),
        in_specs=[
            pl.BlockSpec((BM, C), lambda b, sched: (b, 0)),
            *[_make_wf_spec(q) for q in range(_NS)],
            *[_make_wp_spec(q) for q in range(_NS)],
        ],
        out_specs=pl.BlockSpec((BM, C), lambda b, sched: (b, 0)),
    )
    return pl.pallas_call(
        _ffn_body,
        grid_spec=grid_spec,
        out_shape=jax.ShapeDtypeStruct((S, C), jnp.float32),
        compiler_params=pltpu.CompilerParams(
            dimension_semantics=("arbitrary",)),
    )(sched, xs, *([w_fc] * _NS), *([w_proj] * _NS))


def kernel(x, gate_w, w_fc, w_proj):
    sc_dispatch, sc_combine = _sc_kernels()
    xr = x.reshape(T, C)
    posa, posb, wa, wb, sched, aux = _router(xr, gate_w)
    posa = posa.reshape(T)
    posb = posb.reshape(T)
    xs = sc_dispatch(xr, posa, posb)
    ys = _ffn(sched.reshape(NB), xs, w_fc, w_proj)
    out = sc_combine(ys, posa, posb, wa, wb)
    return out.reshape(x.shape), aux[0, 0]


# speedup vs baseline: 1.1469x; 1.1469x over previous
import functools

import jax
import jax.numpy as jnp
from jax import lax
from jax.experimental import pallas as pl
from jax.experimental.pallas import tpu as pltpu
from jax.experimental.pallas import tpu_sc as plsc

E = 64
AUX_W = 0.01
T, C, DFF = 2048, 768, 3072
DFF_CHUNK = 512
ND = DFF // DFF_CHUNK
BM = 128
NB = T // BM * 2 + E
SCHED_LEN = NB + 8
S = NB * BM
NW = 32
TPW = T // NW
CB = 256


def _router_body(x_ref, gate_ref, posa_ref, posb_ref, wa_ref, wb_ref,
                 sched_ref, aux_ref):
    x = x_ref[...]
    logits = lax.dot_general(x, gate_ref[...], (((1,), (1,)), ((), ())),
                             preferred_element_type=jnp.float32)
    lane = lax.broadcasted_iota(jnp.int32, (T, E), 1)
    m1 = jnp.max(logits, axis=1, keepdims=True)
    i1 = jnp.min(jnp.where(logits == m1, lane, E), axis=1, keepdims=True)
    masked = jnp.where(lane == i1, -jnp.inf, logits)
    m2 = jnp.max(masked, axis=1, keepdims=True)
    i2 = jnp.min(jnp.where(masked == m2, lane, E), axis=1, keepdims=True)
    t = jnp.exp(m2 - m1)
    wa_ref[...] = jnp.broadcast_to(1.0 / (1.0 + t), (T, 16))
    wb_ref[...] = jnp.broadcast_to(t / (1.0 + t), (T, 16))
    p = jnp.exp(logits - m1)
    p = p / jnp.sum(p, axis=1, keepdims=True)
    frac = jnp.mean(p, axis=0, keepdims=True)
    aux_ref[...] = (AUX_W * E * jnp.sum(frac * frac)).reshape(1, 1)

    oh1 = jnp.where(lane == i1, 1.0, 0.0)
    oh2 = jnp.where(lane == i2, 1.0, 0.0)
    ohsum = oh1 + oh2
    rsub = lax.broadcasted_iota(jnp.int32, (CB, CB), 0)
    csub = lax.broadcasted_iota(jnp.int32, (CB, CB), 1)
    tril = jnp.where(csub < rsub, 1.0, 0.0)
    exc_blocks = []
    base = jnp.zeros((1, E), dtype=jnp.float32)
    for rb in range(T // CB):
        mb = ohsum[rb * CB:(rb + 1) * CB, :]
        exc_blocks.append(
            lax.dot_general(tril, mb, (((1,), (0,)), ((), ())),
                            preferred_element_type=jnp.float32) + base)
        base = base + jnp.sum(mb, axis=0, keepdims=True)
    exc = jnp.concatenate(exc_blocks, axis=0)
    counts = base
    nb = jnp.ceil(counts * (1.0 / BM))
    ce1 = lax.broadcasted_iota(jnp.int32, (E, E), 0)
    ce2 = lax.broadcasted_iota(jnp.int32, (E, E), 1)
    upper = jnp.where(ce1 < ce2, 1.0, 0.0)
    bstart = lax.dot_general(nb, upper, (((1,), (0,)), ((), ())),
                             preferred_element_type=jnp.float32)
    sbase = bstart * BM
    posa = jnp.sum(oh1 * (sbase + exc), axis=1, keepdims=True)
    posb = jnp.sum(oh2 * (sbase + exc), axis=1, keepdims=True)
    posa_ref[...] = posa.astype(jnp.int32)
    posb_ref[...] = posb.astype(jnp.int32)
    biota = lax.broadcasted_iota(jnp.int32, (SCHED_LEN, 1), 0).astype(jnp.float32)
    sched = jnp.sum(jnp.where(bstart <= biota, 1.0, 0.0),
                    axis=1, keepdims=True) - 1.0
    total = jnp.sum(nb)
    lane_e = lax.broadcasted_iota(jnp.int32, (1, E), 1).astype(jnp.float32)
    last_used = jnp.max(jnp.where(nb > 0, lane_e, -1.0))
    sched = jnp.where(biota < total, sched, last_used)
    sched = jnp.where(biota < NB, sched, total)
    sched_ref[...] = sched.astype(jnp.int32)


def _router(xr, gate_w):
    return pl.pallas_call(
        _router_body,
        grid=(1,),
        in_specs=[
            pl.BlockSpec((T, C), lambda i: (0, 0)),
            pl.BlockSpec((E, C), lambda i: (0, 0)),
        ],
        out_specs=[
            pl.BlockSpec((T, 1), lambda i: (0, 0)),
            pl.BlockSpec((T, 1), lambda i: (0, 0)),
            pl.BlockSpec((T, 16), lambda i: (0, 0)),
            pl.BlockSpec((T, 16), lambda i: (0, 0)),
            pl.BlockSpec((SCHED_LEN, 1), lambda i: (0, 0)),
            pl.BlockSpec((1, 1), lambda i: (0, 0)),
        ],
        out_shape=[
            jax.ShapeDtypeStruct((T, 1), jnp.int32),
            jax.ShapeDtypeStruct((T, 1), jnp.int32),
            jax.ShapeDtypeStruct((T, 16), jnp.float32),
            jax.ShapeDtypeStruct((T, 16), jnp.float32),
            jax.ShapeDtypeStruct((SCHED_LEN, 1), jnp.int32),
            jax.ShapeDtypeStruct((1, 1), jnp.float32),
        ],
    )(xr, gate_w)


@functools.lru_cache(maxsize=1)
def _sc_kernels():
    mesh = plsc.VectorSubcoreMesh(core_axis_name="c", subcore_axis_name="s")

    @functools.partial(
        pl.kernel,
        mesh=mesh,
        out_type=jax.ShapeDtypeStruct((S, C), jnp.float32),
        scratch_types=[
            pltpu.VMEM((TPW,), jnp.int32),
            pltpu.VMEM((TPW,), jnp.int32),
            pltpu.VMEM((TPW, C), jnp.float32),
            pltpu.SemaphoreType.DMA,
            pltpu.SemaphoreType.DMA,
        ],
    )
    def _sc_dispatch(xr_hbm, posa_hbm, posb_hbm, xs_hbm,
                     posa_v, posb_v, rows_v, sema, semb):
        wid = lax.axis_index("s") * 2 + lax.axis_index("c")
        base = wid * TPW
        pltpu.sync_copy(posa_hbm.at[pl.ds(base, TPW)], posa_v)
        pltpu.sync_copy(posb_hbm.at[pl.ds(base, TPW)], posb_v)
        pltpu.sync_copy(xr_hbm.at[pl.ds(base, TPW)], rows_v)
        cpa = pltpu.async_copy(rows_v, xs_hbm.at[posa_v], sema)
        cpb = pltpu.async_copy(rows_v, xs_hbm.at[posb_v], semb)
        cpa.wait()
        cpb.wait()

    @functools.partial(
        pl.kernel,
        mesh=mesh,
        out_type=jax.ShapeDtypeStruct((T, C), jnp.float32),
        scratch_types=[
            pltpu.VMEM((TPW,), jnp.int32),
            pltpu.VMEM((TPW,), jnp.int32),
            pltpu.VMEM((TPW, 16), jnp.float32),
            pltpu.VMEM((TPW, 16), jnp.float32),
            pltpu.VMEM((TPW, C), jnp.float32),
            pltpu.VMEM((TPW, C), jnp.float32),
            pltpu.SemaphoreType.DMA,
            pltpu.SemaphoreType.DMA,
        ],
    )
    def _sc_combine(ys_hbm, posa_hbm, posb_hbm, wa_hbm, wb_hbm, out_hbm,
                    posa_v, posb_v, wa_v, wb_v, rowsa_v, rowsb_v, sema, semb):
        wid = lax.axis_index("s") * 2 + lax.axis_index("c")
        base = wid * TPW
        pltpu.sync_copy(posa_hbm.at[pl.ds(base, TPW)], posa_v)
        pltpu.sync_copy(posb_hbm.at[pl.ds(base, TPW)], posb_v)
        pltpu.sync_copy(wa_hbm.at[pl.ds(base, TPW)], wa_v)
        pltpu.sync_copy(wb_hbm.at[pl.ds(base, TPW)], wb_v)
        cpa = pltpu.async_copy(ys_hbm.at[posa_v], rowsa_v, sema)
        cpb = pltpu.async_copy(ys_hbm.at[posb_v], rowsb_v, semb)
        cpa.wait()
        cpb.wait()

        def tok_body(ti, carry):
            wa16 = wa_v[ti, :]
            wb16 = wb_v[ti, :]
            for c in range(C // 16):
                ra = rowsa_v[ti, pl.ds(c * 16, 16)]
                rb = rowsb_v[ti, pl.ds(c * 16, 16)]
                rowsa_v[ti, pl.ds(c * 16, 16)] = wa16 * ra + wb16 * rb
            return carry

        lax.fori_loop(0, TPW, tok_body, 0)
        pltpu.sync_copy(rowsa_v, out_hbm.at[pl.ds(base, TPW)])

    return _sc_dispatch, _sc_combine


_NS = 2
_CH = C // _NS
_DH = DFF // _NS


def _ffn_body(sched_ref, xs_ref, *refs):
    wf_refs = refs[:_NS]
    wp_refs = refs[_NS:2 * _NS]
    ys_ref = refs[2 * _NS]
    b = pl.program_id(0)

    @pl.when(b < sched_ref[NB])
    def _compute():
        x = xs_ref[...]
        h = sum(
            lax.dot_general(x[:, q * _CH:(q + 1) * _CH], wf_refs[q][0],
                            (((1,), (0,)), ((), ())),
                            preferred_element_type=jnp.float32)
            for q in range(_NS))
        h = 0.5 * h * (1.0 + lax.erf(h * 0.7071067811865476))
        ys_ref[...] = sum(
            lax.dot_general(h[:, q * _DH:(q + 1) * _DH], wp_refs[q][0],
                            (((1,), (0,)), ((), ())),
                            preferred_element_type=jnp.float32)
            for q in range(_NS))


def _make_wf_spec(q):
    return pl.BlockSpec((1, _CH, DFF), lambda b, sched: (sched[b], q, 0))


def _make_wp_spec(q):
    return pl.BlockSpec((1, _DH, C), lambda b, sched: (sched[b], q, 0))


def _ffn(sched, xs, w_fc, w_proj):
    def _clamp(b, sched):
        return jnp.minimum(b, sched[NB] - 1)

    grid_spec = pltpu.PrefetchScalarGridSpec(
        num_scalar_prefetch=1,
        grid=(NB,),
        in_specs=[
            pl.BlockSpec((BM, C), lambda b, sched: (_clamp(b, sched), 0)),
            *[_make_wf_spec(q) for q in range(_NS)],
            *[_make_wp_spec(q) for q in range(_NS)],
        ],
        out_specs=pl.BlockSpec((BM, C), lambda b, sched: (_clamp(b, sched), 0)),
    )
    return pl.pallas_call(
        _ffn_body,
        grid_spec=grid_spec,
        out_shape=jax.ShapeDtypeStruct((S, C), jnp.float32),
        compiler_params=pltpu.CompilerParams(
            dimension_semantics=("arbitrary",)),
    )(sched, xs, *([w_fc] * _NS), *([w_proj] * _NS))


def kernel(x, gate_w, w_fc, w_proj):
    sc_dispatch, sc_combine = _sc_kernels()
    xr = x.reshape(T, C)
    posa, posb, wa, wb, sched, aux = _router(xr, gate_w)
    posa = posa.reshape(T)
    posb = posb.reshape(T)
    xs = sc_dispatch(xr, posa, posb)
    ys = _ffn(sched.reshape(SCHED_LEN), xs, w_fc, w_proj)
    out = sc_combine(ys, posa, posb, wa, wb)
    return out.reshape(x.shape), aux[0, 0]


# speedup vs baseline: 1.1504x; 1.0030x over previous
import functools

import jax
import jax.numpy as jnp
from jax import lax
from jax.experimental import pallas as pl
from jax.experimental.pallas import tpu as pltpu
from jax.experimental.pallas import tpu_sc as plsc

E = 64
AUX_W = 0.01
T, C, DFF = 2048, 768, 3072
BM = 128
NB = T // BM * 2 + E
SCHED_LEN = NB + 8
S = NB * BM
NW = 32
TPW = T // NW
CB = 256


def _router_body(x_ref, gate_ref, posa_ref, posb_ref, wa_ref, wb_ref,
                 sched_ref, aux_ref):
    x = x_ref[...]
    logits = lax.dot_general(x, gate_ref[...], (((1,), (1,)), ((), ())),
                             preferred_element_type=jnp.float32)
    lane = lax.broadcasted_iota(jnp.int32, (T, E), 1)
    m1 = jnp.max(logits, axis=1, keepdims=True)
    i1 = jnp.min(jnp.where(logits == m1, lane, E), axis=1, keepdims=True)
    masked = jnp.where(lane == i1, -jnp.inf, logits)
    m2 = jnp.max(masked, axis=1, keepdims=True)
    i2 = jnp.min(jnp.where(masked == m2, lane, E), axis=1, keepdims=True)
    t = jnp.exp(m2 - m1)
    wa_ref[...] = jnp.broadcast_to(1.0 / (1.0 + t), (T, 16))
    wb_ref[...] = jnp.broadcast_to(t / (1.0 + t), (T, 16))
    p = jnp.exp(logits - m1)
    p = p / jnp.sum(p, axis=1, keepdims=True)
    frac = jnp.mean(p, axis=0, keepdims=True)
    aux_ref[...] = (AUX_W * E * jnp.sum(frac * frac)).reshape(1, 1)

    oh1 = jnp.where(lane == i1, 1.0, 0.0)
    oh2 = jnp.where(lane == i2, 1.0, 0.0)
    ohsum = oh1 + oh2
    rsub = lax.broadcasted_iota(jnp.int32, (CB, CB), 0)
    csub = lax.broadcasted_iota(jnp.int32, (CB, CB), 1)
    tril = jnp.where(csub < rsub, 1.0, 0.0)
    exc_blocks = []
    base = jnp.zeros((1, E), dtype=jnp.float32)
    for rb in range(T // CB):
        mb = ohsum[rb * CB:(rb + 1) * CB, :]
        exc_blocks.append(
            lax.dot_general(tril, mb, (((1,), (0,)), ((), ())),
                            preferred_element_type=jnp.float32) + base)
        base = base + jnp.sum(mb, axis=0, keepdims=True)
    exc = jnp.concatenate(exc_blocks, axis=0)
    counts = base
    nb = jnp.ceil(counts * (1.0 / BM))
    ce1 = lax.broadcasted_iota(jnp.int32, (E, E), 0)
    ce2 = lax.broadcasted_iota(jnp.int32, (E, E), 1)
    upper = jnp.where(ce1 < ce2, 1.0, 0.0)
    bstart = lax.dot_general(nb, upper, (((1,), (0,)), ((), ())),
                             preferred_element_type=jnp.float32)
    sbase = bstart * BM
    posa = jnp.sum(oh1 * (sbase + exc), axis=1, keepdims=True)
    posb = jnp.sum(oh2 * (sbase + exc), axis=1, keepdims=True)
    posa_ref[...] = posa.astype(jnp.int32)
    posb_ref[...] = posb.astype(jnp.int32)
    biota = lax.broadcasted_iota(jnp.int32, (SCHED_LEN, 1), 0).astype(jnp.float32)
    sched = jnp.sum(jnp.where(bstart <= biota, 1.0, 0.0),
                    axis=1, keepdims=True) - 1.0
    total = jnp.sum(nb)
    lane_e = lax.broadcasted_iota(jnp.int32, (1, E), 1).astype(jnp.float32)
    last_used = jnp.max(jnp.where(nb > 0, lane_e, -1.0))
    sched = jnp.where(biota < total, sched, last_used)
    sched = jnp.where(biota < NB, sched, total)
    sched_ref[...] = sched.astype(jnp.int32)


def _router(xr, gate_w):
    return pl.pallas_call(
        _router_body,
        grid=(1,),
        in_specs=[
            pl.BlockSpec((T, C), lambda i: (0, 0)),
            pl.BlockSpec((E, C), lambda i: (0, 0)),
        ],
        out_specs=[
            pl.BlockSpec((T, 1), lambda i: (0, 0)),
            pl.BlockSpec((T, 1), lambda i: (0, 0)),
            pl.BlockSpec((T, 16), lambda i: (0, 0)),
            pl.BlockSpec((T, 16), lambda i: (0, 0)),
            pl.BlockSpec((SCHED_LEN, 1), lambda i: (0, 0)),
            pl.BlockSpec((1, 1), lambda i: (0, 0)),
        ],
        out_shape=[
            jax.ShapeDtypeStruct((T, 1), jnp.int32),
            jax.ShapeDtypeStruct((T, 1), jnp.int32),
            jax.ShapeDtypeStruct((T, 16), jnp.float32),
            jax.ShapeDtypeStruct((T, 16), jnp.float32),
            jax.ShapeDtypeStruct((SCHED_LEN, 1), jnp.int32),
            jax.ShapeDtypeStruct((1, 1), jnp.float32),
        ],
    )(xr, gate_w)


@functools.lru_cache(maxsize=1)
def _sc_kernels():
    mesh = plsc.VectorSubcoreMesh(core_axis_name="c", subcore_axis_name="s")

    @functools.partial(
        pl.kernel,
        mesh=mesh,
        out_type=jax.ShapeDtypeStruct((S, C), jnp.float32),
        scratch_types=[
            pltpu.VMEM((TPW,), jnp.int32),
            pltpu.VMEM((TPW,), jnp.int32),
            pltpu.VMEM((TPW, C), jnp.float32),
            pltpu.SemaphoreType.DMA,
            pltpu.SemaphoreType.DMA,
        ],
    )
    def _sc_dispatch(xr_hbm, posa_hbm, posb_hbm, xs_hbm,
                     posa_v, posb_v, rows_v, sema, semb):
        wid = lax.axis_index("s") * 2 + lax.axis_index("c")
        base = wid * TPW
        pltpu.sync_copy(posa_hbm.at[pl.ds(base, TPW)], posa_v)
        pltpu.sync_copy(posb_hbm.at[pl.ds(base, TPW)], posb_v)
        pltpu.sync_copy(xr_hbm.at[pl.ds(base, TPW)], rows_v)
        cpa = pltpu.async_copy(rows_v, xs_hbm.at[posa_v], sema)
        cpb = pltpu.async_copy(rows_v, xs_hbm.at[posb_v], semb)
        cpa.wait()
        cpb.wait()

    @functools.partial(
        pl.kernel,
        mesh=mesh,
        out_type=jax.ShapeDtypeStruct((T, C), jnp.float32),
        scratch_types=[
            pltpu.VMEM((TPW,), jnp.int32),
            pltpu.VMEM((TPW,), jnp.int32),
            pltpu.VMEM((TPW, 16), jnp.float32),
            pltpu.VMEM((TPW, 16), jnp.float32),
            pltpu.VMEM((TPW, C), jnp.float32),
            pltpu.VMEM((TPW, C), jnp.float32),
            pltpu.SemaphoreType.DMA,
            pltpu.SemaphoreType.DMA,
        ],
    )
    def _sc_combine(ys_hbm, posa_hbm, posb_hbm, wa_hbm, wb_hbm, out_hbm,
                    posa_v, posb_v, wa_v, wb_v, rowsa_v, rowsb_v, sema, semb):
        wid = lax.axis_index("s") * 2 + lax.axis_index("c")
        base = wid * TPW
        pltpu.sync_copy(posa_hbm.at[pl.ds(base, TPW)], posa_v)
        pltpu.sync_copy(posb_hbm.at[pl.ds(base, TPW)], posb_v)
        pltpu.sync_copy(wa_hbm.at[pl.ds(base, TPW)], wa_v)
        pltpu.sync_copy(wb_hbm.at[pl.ds(base, TPW)], wb_v)
        cpa = pltpu.async_copy(ys_hbm.at[posa_v], rowsa_v, sema)
        cpb = pltpu.async_copy(ys_hbm.at[posb_v], rowsb_v, semb)
        cpa.wait()
        cpb.wait()

        def tok_body(ti, carry):
            wa16 = wa_v[ti, :]
            wb16 = wb_v[ti, :]
            for c in range(C // 16):
                ra = rowsa_v[ti, pl.ds(c * 16, 16)]
                rb = rowsb_v[ti, pl.ds(c * 16, 16)]
                rowsa_v[ti, pl.ds(c * 16, 16)] = wa16 * ra + wb16 * rb
            return carry

        lax.fori_loop(0, TPW, tok_body, 0)
        pltpu.sync_copy(rowsa_v, out_hbm.at[pl.ds(base, TPW)])

    return _sc_dispatch, _sc_combine


_NS = 2
_CH = C // _NS
_DH = DFF // _NS


def _ffn_body(sched_ref, xs_ref, *refs):
    wf_refs = refs[:_NS]
    wp_refs = refs[_NS:2 * _NS]
    ys_ref = refs[2 * _NS]
    b = pl.program_id(0)

    @pl.when(b < sched_ref[NB])
    def _compute():
        x = xs_ref[...]
        h = sum(
            lax.dot_general(x[:, q * _CH:(q + 1) * _CH], wf_refs[q][0],
                            (((1,), (0,)), ((), ())),
                            preferred_element_type=jnp.float32)
            for q in range(_NS))
        h = 0.5 * h * (1.0 + lax.erf(h * 0.7071067811865476))
        ys_ref[...] = sum(
            lax.dot_general(h[:, q * _DH:(q + 1) * _DH], wp_refs[q][0],
                            (((1,), (0,)), ((), ())),
                            preferred_element_type=jnp.float32)
            for q in range(_NS))


def _make_wf_spec(q):
    return pl.BlockSpec((1, _CH, DFF), lambda b, sched: (sched[b], q, 0))


def _make_wp_spec(q):
    return pl.BlockSpec((1, _DH, C), lambda b, sched: (sched[b], q, 0))


def _ffn(sched, xs, w_fc, w_proj):
    def _clamp(b, sched):
        return jnp.minimum(b, sched[NB] - 1)

    grid_spec = pltpu.PrefetchScalarGridSpec(
        num_scalar_prefetch=1,
        grid=(NB,),
        in_specs=[
            pl.BlockSpec((BM, C), lambda b, sched: (_clamp(b, sched), 0)),
            *[_make_wf_spec(q) for q in range(_NS)],
            *[_make_wp_spec(q) for q in range(_NS)],
        ],
        out_specs=pl.BlockSpec((BM, C), lambda b, sched: (_clamp(b, sched), 0)),
    )
    return pl.pallas_call(
        _ffn_body,
        grid_spec=grid_spec,
        out_shape=jax.ShapeDtypeStruct((S, C), jnp.float32),
        compiler_params=pltpu.CompilerParams(
            dimension_semantics=("arbitrary",)),
    )(sched, xs, *([w_fc] * _NS), *([w_proj] * _NS))


def kernel(x, gate_w, w_fc, w_proj):
    sc_dispatch, sc_combine = _sc_kernels()
    xr = x.reshape(T, C)
    posa, posb, wa, wb, sched, aux = _router(xr, gate_w)
    posa = posa.reshape(T)
    posb = posb.reshape(T)
    xs = sc_dispatch(xr, posa, posb)
    ys = _ffn(sched.reshape(SCHED_LEN), xs, w_fc, w_proj)
    out = sc_combine(ys, posa, posb, wa, wb)
    return out.reshape(x.shape), aux[0, 0]
